# baseline (speedup 1.0000x reference)
import functools
import math

import jax
import jax.numpy as jnp
import numpy as np
from jax import lax
from jax.experimental import pallas as pl
from jax.experimental.pallas import tpu as pltpu
from jax.experimental.pallas import tpu_sc as plsc

_N_NODES = 10000
_E = 160000
_NF = 128
_NRBF = 20
_RCUT = 5.0

_B = 640
_NT = 16
_CHUNK = 80
_PER_TILE = _E // _NT
_NCHUNK = _PER_TILE // _CHUNK
_ROWS_T = 624
_TAIL = _N_NODES - _NT * _ROWS_T
_ZROWS = 208


_E3_NP = np.zeros((8, 3 * _NF), np.float32)
for _g in range(3):
    _E3_NP[_g, _g * _NF:(_g + 1) * _NF] = 1.0


def _dot(a, b):
    return jnp.dot(a, b, preferred_element_type=jnp.float32)


def _tc_body(sj, vjT, rpadT, W1, b1, W2, b2, Wrpa, E3,
             out0, out1, out2, out3):
    rT = rpadT[...]
    sqT = jnp.sum(rT * rT, axis=0, keepdims=True)
    rnT = jnp.sqrt(sqT)
    invT = 1.0 / (rnT + 1e-8)
    fcutT = jnp.where(rnT > _RCUT, 0.0,
                      0.5 * (jnp.cos((math.pi / _RCUT) * rnT) + 1.0))
    kcol = (lax.broadcasted_iota(jnp.int32, (32, 1), 0) + 1
            ).astype(jnp.float32)
    args = kcol * ((math.pi / _RCUT) * rnT)
    rbfT = jnp.sin(args) * (invT * fcutT)
    rowid = lax.broadcasted_iota(jnp.int32, (32, _B), 0)
    rbfT = jnp.where(rowid == _NRBF, fcutT, rbfT)
    rbf = rbfT.T
    ws = _dot(rbf, Wrpa[...])
    rhat = (rT * invT).T
    h = _dot(sj[...], W1[...]) + b1[...]
    h = h * jax.nn.sigmoid(h)
    phi = _dot(h, W2[...]) + b2[...]
    phiw = phi * ws
    s1 = phiw[:, :_NF]
    s2 = phiw[:, _NF:2 * _NF]
    s3 = phiw[:, 2 * _NF:]
    rhe = _dot(rhat, E3[...])
    vt = vjT[...]
    out0[...] = vt[0] * s1 + rhe[:, :_NF] * s3
    out1[...] = vt[1] * s1 + rhe[:, _NF:2 * _NF] * s3
    out2[...] = vt[2] * s1 + rhe[:, 2 * _NF:] * s3
    out3[...] = s2


def _make_tc_call(block_off, nblocks):
    edge_spec = pl.BlockSpec((_B, _NF), lambda i: (i + block_off, 0))
    out_spec = pl.BlockSpec((_B, _NF), lambda i: (i, 0))
    return pl.pallas_call(
        _tc_body,
        grid=(nblocks,),
        in_specs=[
            edge_spec,
            pl.BlockSpec((3, _B, _NF),
                         lambda i: (0, i + block_off, 0)),
            pl.BlockSpec((8, _B), lambda i: (0, i + block_off)),
            pl.BlockSpec((_NF, _NF), lambda i: (0, 0)),
            pl.BlockSpec((1, _NF), lambda i: (0, 0)),
            pl.BlockSpec((_NF, 3 * _NF), lambda i: (0, 0)),
            pl.BlockSpec((1, 3 * _NF), lambda i: (0, 0)),
            pl.BlockSpec((32, 3 * _NF), lambda i: (0, 0)),
            pl.BlockSpec((8, 3 * _NF), lambda i: (0, 0)),
        ],
        out_specs=[out_spec, out_spec, out_spec, out_spec],
        out_shape=[jax.ShapeDtypeStruct((nblocks * _B, _NF), jnp.float32)] * 4,
    )


_NB_HALF = _E // _B // 2
_E_HALF = _E // 2
_tc_call_a = _make_tc_call(0, _NB_HALF)
_tc_call_b = _make_tc_call(_NB_HALF, _NB_HALF)


def _sc_body_impl(crefs, dsts, inits, outs,
                  buf0, buf1, idx0, idx1, zbuf, acc, sem, sem2,
                  edge_off, n_edges, chunk):
    c = lax.axis_index("c")
    s = lax.axis_index("s")
    bufs = [buf0, buf1]
    idxs = [idx0, idx1]
    per_tile = n_edges // _NT
    nchunk = per_tile // chunk

    if inits is None:
        def zrow(i, carry):
            def zcol(j, carry2):
                zbuf[i, pl.ds(j * 16, 16)] = jnp.zeros((16,), jnp.float32)
                return carry2
            return lax.fori_loop(0, 8, zcol, carry)
        lax.fori_loop(0, _ZROWS, zrow, 0)

    row0 = pl.multiple_of(s * _ROWS_T, 8)
    cbase = pl.multiple_of(s * per_tile, 8)
    dbase = pl.multiple_of(edge_off + s * per_tile, 8)
    tail0 = _NT * _ROWS_T

    for gi in range(4):
        @pl.when(c == gi // 2)
        def _round(gi=gi):
            cref = crefs[gi]
            outg = outs[gi]

            for i in range(_ROWS_T // _ZROWS):
                rr = pl.multiple_of(row0 + i * _ZROWS, 8)
                if inits is None:
                    pltpu.sync_copy(zbuf, acc.at[pl.ds(rr, _ZROWS)])
                else:
                    pltpu.sync_copy(inits[gi].at[pl.ds(rr, _ZROWS)],
                                    acc.at[pl.ds(rr, _ZROWS)])

            @pl.when(s == _NT - 1)
            def _seed_tail():
                if inits is None:
                    pltpu.sync_copy(zbuf.at[pl.ds(0, _TAIL)],
                                    acc.at[pl.ds(tail0, _TAIL)])
                else:
                    pltpu.sync_copy(inits[gi].at[pl.ds(tail0, _TAIL)],
                                    acc.at[pl.ds(tail0, _TAIL)])
            plsc.subcore_barrier()

            def _start(j, slot):
                d0 = pl.multiple_of(dbase + j * chunk, 8)
                e0 = pl.multiple_of(cbase + j * chunk, 8)
                pltpu.async_copy(dsts.at[pl.ds(d0, chunk)], idxs[slot], sem)
                pltpu.async_copy(cref.at[pl.ds(e0, chunk)], bufs[slot], sem)

            def _drain(j, slot):
                d0 = pl.multiple_of(dbase + j * chunk, 8)
                e0 = pl.multiple_of(cbase + j * chunk, 8)
                pltpu.make_async_copy(dsts.at[pl.ds(d0, chunk)],
                                      idxs[slot], sem).wait()
                pltpu.make_async_copy(cref.at[pl.ds(e0, chunk)],
                                      bufs[slot], sem).wait()

            def _wait_scatter(slot):
                pltpu.make_async_copy(bufs[slot], acc.at[idxs[slot]],
                                      sem2).wait()

            _start(0, 0)

            def outer(j2, carry):
                for b in range(2):
                    j = j2 + b

                    @pl.when(j < nchunk)
                    def _step(j=j, b=b):
                        _drain(j, b)

                        @pl.when(j > 0)
                        def _wprev():
                            _wait_scatter(1 - b)

                        @pl.when(j + 1 < nchunk)
                        def _prefetch():
                            _start(j + 1, 1 - b)
                        pltpu.async_copy(bufs[b], acc.at[idxs[b]], sem2,
                                         add=True)
                return carry
            lax.fori_loop(0, (nchunk + 1) // 2, lambda t, cr: outer(t * 2, cr),
                          0)
            _wait_scatter((nchunk - 1) % 2)
            plsc.subcore_barrier()

            for i in range(_ROWS_T // _ZROWS):
                rr = pl.multiple_of(row0 + i * _ZROWS, 8)
                pltpu.sync_copy(acc.at[pl.ds(rr, _ZROWS)],
                                outg.at[pl.ds(rr, _ZROWS)])

            @pl.when(s == _NT - 1)
            def _write_tail():
                pltpu.sync_copy(acc.at[pl.ds(tail0, _TAIL)],
                                outg.at[pl.ds(tail0, _TAIL)])


@functools.cache
def _sc_call(edge_off, n_edges, chunk, with_init):
    if with_init:
        def body(c0, c1, c2, c3, dsts, i0, i1, i2, i3,
                 o0, o1, o2, o3, *scratch):
            _sc_body_impl([c0, c1, c2, c3], dsts, [i0, i1, i2, i3],
                          [o0, o1, o2, o3], *scratch,
                          edge_off=edge_off, n_edges=n_edges, chunk=chunk)
    else:
        def body(c0, c1, c2, c3, dsts,
                 o0, o1, o2, o3, *scratch):
            _sc_body_impl([c0, c1, c2, c3], dsts, None,
                          [o0, o1, o2, o3], *scratch,
                          edge_off=edge_off, n_edges=n_edges, chunk=chunk)
    return pl.kernel(
        body,
        out_type=[jax.ShapeDtypeStruct((_N_NODES, _NF), jnp.float32)] * 4,
        mesh=plsc.VectorSubcoreMesh(core_axis_name="c", subcore_axis_name="s"),
        scratch_types=[
            pltpu.VMEM((chunk, _NF), jnp.float32),
            pltpu.VMEM((chunk, _NF), jnp.float32),
            pltpu.VMEM((chunk,), jnp.int32),
            pltpu.VMEM((chunk,), jnp.int32),
            pltpu.VMEM((_ZROWS, _NF), jnp.float32),
            pltpu.VMEM_SHARED((_N_NODES, _NF), jnp.float32),
            pltpu.SemaphoreType.DMA,
            pltpu.SemaphoreType.DMA,
        ],
        compiler_params=pltpu.CompilerParams(use_tc_tiling_on_sc=True),
    )


def kernel(vj, sj, rij_vec, eij, W1, b1, W2, b2, Wr, br):
    vjT = jnp.transpose(vj, (2, 0, 1))
    rpadT = jnp.pad(rij_vec.T, ((0, 5), (0, 0)))
    Wrpa = jnp.concatenate(
        [Wr, br.reshape(1, 3 * _NF),
         jnp.zeros((32 - _NRBF - 1, 3 * _NF), jnp.float32)], axis=0)
    dense_args = (W1, b1.reshape(1, _NF), W2, b2.reshape(1, 3 * _NF), Wrpa,
                  jnp.asarray(_E3_NP))
    dst = eij[1]
    ca = _tc_call_a(sj, vjT, rpadT, *dense_args)
    cb = _tc_call_b(sj, vjT, rpadT, *dense_args)
    pa = _sc_call(0, _E_HALF, 40, False)(*ca, dst)
    o0, o1, o2, o3 = _sc_call(_E_HALF, _E_HALF, 40, True)(*cb, dst, *pa)
    d_vim = jnp.stack([o0, o1, o2], axis=-1)
    d_sim = o3
    return (d_vim, d_sim)


# speedup vs baseline: 1.3835x; 1.3835x over previous
import functools
import math

import jax
import jax.numpy as jnp
import numpy as np
from jax import lax
from jax.experimental import pallas as pl
from jax.experimental.pallas import tpu as pltpu
from jax.experimental.pallas import tpu_sc as plsc

_N_NODES = 10000
_E = 160000
_NF = 128
_NRBF = 20
_RCUT = 5.0

_B = 640
_NT = 16
_CHUNK = 80
_PER_TILE = _E // _NT
_NCHUNK = _PER_TILE // _CHUNK
_ROWS_T = 624
_TAIL = _N_NODES - _NT * _ROWS_T
_ZROWS = 208


_E3_NP = np.zeros((8, 3 * _NF), np.float32)
for _g in range(3):
    _E3_NP[_g, _g * _NF:(_g + 1) * _NF] = 1.0


def _dot(a, b):
    return jnp.dot(a, b, preferred_element_type=jnp.float32)


def _tc_body(sj, vjT, rpadT, W1, b1, W2, b2, Wrpa, E3,
             out0, out1, out2, out3):
    rT = rpadT[...]
    sqT = jnp.sum(rT * rT, axis=0, keepdims=True)
    rnT = jnp.sqrt(sqT)
    invT = 1.0 / (rnT + 1e-8)
    fcutT = jnp.where(rnT > _RCUT, 0.0,
                      0.5 * (jnp.cos((math.pi / _RCUT) * rnT) + 1.0))
    kcol = (lax.broadcasted_iota(jnp.int32, (32, 1), 0) + 1
            ).astype(jnp.float32)
    args = kcol * ((math.pi / _RCUT) * rnT)
    rbfT = jnp.sin(args) * (invT * fcutT)
    rowid = lax.broadcasted_iota(jnp.int32, (32, _B), 0)
    rbfT = jnp.where(rowid == _NRBF, fcutT, rbfT)
    rbf = rbfT.T
    ws = _dot(rbf, Wrpa[...])
    rhat = (rT * invT).T
    h = _dot(sj[...], W1[...]) + b1[...]
    h = h * jax.nn.sigmoid(h)
    phi = _dot(h, W2[...]) + b2[...]
    phiw = phi * ws
    s1 = phiw[:, :_NF]
    s2 = phiw[:, _NF:2 * _NF]
    s3 = phiw[:, 2 * _NF:]
    rhe = _dot(rhat, E3[...])
    vt = vjT[...]
    out0[...] = vt[0] * s1 + rhe[:, :_NF] * s3
    out1[...] = vt[1] * s1 + rhe[:, _NF:2 * _NF] * s3
    out2[...] = vt[2] * s1 + rhe[:, 2 * _NF:] * s3
    out3[...] = s2


def _make_tc_call(block_off, nblocks):
    edge_spec = pl.BlockSpec((_B, _NF), lambda i: (i + block_off, 0))
    out_spec = pl.BlockSpec((_B, _NF), lambda i: (i, 0))
    return pl.pallas_call(
        _tc_body,
        grid=(nblocks,),
        in_specs=[
            edge_spec,
            pl.BlockSpec((3, _B, _NF),
                         lambda i: (0, i + block_off, 0)),
            pl.BlockSpec((8, _B), lambda i: (0, i + block_off)),
            pl.BlockSpec((_NF, _NF), lambda i: (0, 0)),
            pl.BlockSpec((1, _NF), lambda i: (0, 0)),
            pl.BlockSpec((_NF, 3 * _NF), lambda i: (0, 0)),
            pl.BlockSpec((1, 3 * _NF), lambda i: (0, 0)),
            pl.BlockSpec((32, 3 * _NF), lambda i: (0, 0)),
            pl.BlockSpec((8, 3 * _NF), lambda i: (0, 0)),
        ],
        out_specs=[out_spec, out_spec, out_spec, out_spec],
        out_shape=[jax.ShapeDtypeStruct((nblocks * _B, _NF), jnp.float32)] * 4,
    )


_NB_HALF = _E // _B // 2
_E_HALF = _E // 2
_tc_call_a = _make_tc_call(0, _NB_HALF)
_tc_call_b = _make_tc_call(_NB_HALF, _NB_HALF)


_DEPTH = 4


def _sc_body_impl(crefs, dsts, inits, outs,
                  buf0, buf1, buf2, buf3, idx0, idx1, idx2, idx3,
                  zbuf, acc, sem, sem2,
                  edge_off, n_edges, chunk):
    c = lax.axis_index("c")
    s = lax.axis_index("s")
    bufs = [buf0, buf1, buf2, buf3]
    idxs = [idx0, idx1, idx2, idx3]
    per_tile = n_edges // _NT
    nchunk = per_tile // chunk

    if inits is None:
        def zrow(i, carry):
            def zcol(j, carry2):
                zbuf[i, pl.ds(j * 16, 16)] = jnp.zeros((16,), jnp.float32)
                return carry2
            return lax.fori_loop(0, 8, zcol, carry)
        lax.fori_loop(0, _ZROWS, zrow, 0)

    row0 = pl.multiple_of(s * _ROWS_T, 8)
    cbase = pl.multiple_of(s * per_tile, 8)
    dbase = pl.multiple_of(edge_off + s * per_tile, 8)
    tail0 = _NT * _ROWS_T

    for gi in range(4):
        @pl.when(c == gi // 2)
        def _round(gi=gi):
            cref = crefs[gi]
            outg = outs[gi]

            for i in range(_ROWS_T // _ZROWS):
                rr = pl.multiple_of(row0 + i * _ZROWS, 8)
                if inits is None:
                    pltpu.sync_copy(zbuf, acc.at[pl.ds(rr, _ZROWS)])
                else:
                    pltpu.sync_copy(inits[gi].at[pl.ds(rr, _ZROWS)],
                                    acc.at[pl.ds(rr, _ZROWS)])

            @pl.when(s == _NT - 1)
            def _seed_tail():
                if inits is None:
                    pltpu.sync_copy(zbuf.at[pl.ds(0, _TAIL)],
                                    acc.at[pl.ds(tail0, _TAIL)])
                else:
                    pltpu.sync_copy(inits[gi].at[pl.ds(tail0, _TAIL)],
                                    acc.at[pl.ds(tail0, _TAIL)])
            plsc.subcore_barrier()

            def _start(j, slot):
                d0 = pl.multiple_of(dbase + j * chunk, 8)
                e0 = pl.multiple_of(cbase + j * chunk, 8)
                pltpu.async_copy(dsts.at[pl.ds(d0, chunk)], idxs[slot], sem)
                pltpu.async_copy(cref.at[pl.ds(e0, chunk)], bufs[slot], sem)

            def _drain(j, slot):
                d0 = pl.multiple_of(dbase + j * chunk, 8)
                e0 = pl.multiple_of(cbase + j * chunk, 8)
                pltpu.make_async_copy(dsts.at[pl.ds(d0, chunk)],
                                      idxs[slot], sem).wait()
                pltpu.make_async_copy(cref.at[pl.ds(e0, chunk)],
                                      bufs[slot], sem).wait()

            def _wait_scatter(slot):
                pltpu.make_async_copy(bufs[slot], acc.at[idxs[slot]],
                                      sem2).wait()

            for p in range(_DEPTH - 1):
                if p < nchunk:
                    _start(p, p)

            def outer(j2, carry):
                for b in range(_DEPTH):
                    j = j2 + b

                    @pl.when(j < nchunk)
                    def _step(j=j, b=b):
                        _drain(j, b)

                        @pl.when(j > 0)
                        def _wprev():
                            _wait_scatter((b - 1) % _DEPTH)

                        @pl.when(j + _DEPTH - 1 < nchunk)
                        def _prefetch():
                            _start(j + _DEPTH - 1, (b - 1) % _DEPTH)
                        pltpu.async_copy(bufs[b], acc.at[idxs[b]], sem2,
                                         add=True)
                return carry
            nouter = (nchunk + _DEPTH - 1) // _DEPTH
            lax.fori_loop(0, nouter, lambda t, cr: outer(t * _DEPTH, cr), 0)
            _wait_scatter((nchunk - 1) % _DEPTH)
            plsc.subcore_barrier()

            for i in range(_ROWS_T // _ZROWS):
                rr = pl.multiple_of(row0 + i * _ZROWS, 8)
                pltpu.sync_copy(acc.at[pl.ds(rr, _ZROWS)],
                                outg.at[pl.ds(rr, _ZROWS)])

            @pl.when(s == _NT - 1)
            def _write_tail():
                pltpu.sync_copy(acc.at[pl.ds(tail0, _TAIL)],
                                outg.at[pl.ds(tail0, _TAIL)])


@functools.cache
def _sc_call(edge_off, n_edges, chunk, with_init):
    if with_init:
        def body(c0, c1, c2, c3, dsts, i0, i1, i2, i3,
                 o0, o1, o2, o3, *scratch):
            _sc_body_impl([c0, c1, c2, c3], dsts, [i0, i1, i2, i3],
                          [o0, o1, o2, o3], *scratch,
                          edge_off=edge_off, n_edges=n_edges, chunk=chunk)
    else:
        def body(c0, c1, c2, c3, dsts,
                 o0, o1, o2, o3, *scratch):
            _sc_body_impl([c0, c1, c2, c3], dsts, None,
                          [o0, o1, o2, o3], *scratch,
                          edge_off=edge_off, n_edges=n_edges, chunk=chunk)
    return pl.kernel(
        body,
        out_type=[jax.ShapeDtypeStruct((_N_NODES, _NF), jnp.float32)] * 4,
        mesh=plsc.VectorSubcoreMesh(core_axis_name="c", subcore_axis_name="s"),
        scratch_types=[
            pltpu.VMEM((chunk, _NF), jnp.float32),
            pltpu.VMEM((chunk, _NF), jnp.float32),
            pltpu.VMEM((chunk, _NF), jnp.float32),
            pltpu.VMEM((chunk, _NF), jnp.float32),
            pltpu.VMEM((chunk,), jnp.int32),
            pltpu.VMEM((chunk,), jnp.int32),
            pltpu.VMEM((chunk,), jnp.int32),
            pltpu.VMEM((chunk,), jnp.int32),
            pltpu.VMEM((_ZROWS, _NF), jnp.float32),
            pltpu.VMEM_SHARED((_N_NODES, _NF), jnp.float32),
            pltpu.SemaphoreType.DMA,
            pltpu.SemaphoreType.DMA,
        ],
        compiler_params=pltpu.CompilerParams(use_tc_tiling_on_sc=True),
    )


def kernel(vj, sj, rij_vec, eij, W1, b1, W2, b2, Wr, br):
    vjT = jnp.transpose(vj, (2, 0, 1))
    rpadT = jnp.pad(rij_vec.T, ((0, 5), (0, 0)))
    Wrpa = jnp.concatenate(
        [Wr, br.reshape(1, 3 * _NF),
         jnp.zeros((32 - _NRBF - 1, 3 * _NF), jnp.float32)], axis=0)
    dense_args = (W1, b1.reshape(1, _NF), W2, b2.reshape(1, 3 * _NF), Wrpa,
                  jnp.asarray(_E3_NP))
    dst = eij[1]
    ca = _tc_call_a(sj, vjT, rpadT, *dense_args)
    cb = _tc_call_b(sj, vjT, rpadT, *dense_args)
    pa = _sc_call(0, _E_HALF, 40, False)(*ca, dst)
    o0, o1, o2, o3 = _sc_call(_E_HALF, _E_HALF, 40, True)(*cb, dst, *pa)
    d_vim = jnp.stack([o0, o1, o2], axis=-1)
    d_sim = o3
    return (d_vim, d_sim)
